# baseline (speedup 1.0000x reference)
import functools

import jax
import jax.numpy as jnp
from jax import lax
from jax.experimental import pallas as pl
from jax.experimental.pallas import tpu as pltpu
from jax.experimental.pallas import tpu_sc as plsc

N_NODES = 10000
N_EDGES = 320000
D = 128
NC = 2
NS = 16
NW = NC * NS
EPT = N_EDGES // NW
CH = 96
K = 106
EPP = K * CH
NP = 10112
RPT = NP // NS


def _elu_body(g_ref, w_ref, out_ref):
    x = g_ref[...] * w_ref[...]
    out_ref[...] = jnp.where(x > 0, x, jnp.exp(jnp.minimum(x, 0.0)) - 1.0)


def _add_body(p_ref, out_ref):
    out_ref[...] = p_ref[0, :N_NODES] + p_ref[1, :N_NODES]


def _sc_body(src_hbm, dst_hbm, emb_hbm, out_hbm, src_v, dst_v, rows0,
             rows1, acc, sem0, sem1):
    cid = lax.axis_index("c")
    sid = lax.axis_index("s")
    wid = cid * NS + sid

    def zstore(t, carry):
        i = t // (D // 16)
        j = t % (D // 16)
        rows0[i, pl.ds(j * 16, 16)] = jnp.zeros((16,), jnp.float32)
        return carry

    lax.fori_loop(0, CH * (D // 16), zstore, 0)

    row0 = sid * RPT
    for r in range(RPT // CH):
        pltpu.sync_copy(rows0, acc.at[pl.ds(row0 + r * CH, CH)])
    rem = RPT % CH
    if rem:
        pltpu.sync_copy(rows0.at[pl.ds(0, rem)],
                        acc.at[pl.ds(row0 + RPT - rem, rem)])
    plsc.subcore_barrier()

    pltpu.sync_copy(src_hbm.at[pl.ds(wid * EPP, EPP)], src_v)
    pltpu.sync_copy(dst_hbm.at[wid], dst_v)

    bufs = ((rows0, sem0), (rows1, sem1))
    for m, (rv, sm) in enumerate(bufs):
        pltpu.async_copy(emb_hbm.at[src_v.at[pl.ds(m * CH, CH)]], rv, sm)

    def pair(g, carry):
        j0 = 2 * g
        for b, (rv, sm) in enumerate(bufs):
            j = j0 + b
            pltpu.make_async_copy(emb_hbm.at[pl.ds(0, CH)], rv, sm).wait()
            pltpu.sync_copy(rv, acc.at[dst_v.at[j]], add=True)

            @pl.when(j + 2 < K)
            def _():
                pltpu.async_copy(
                    emb_hbm.at[src_v.at[pl.ds((j + 2) * CH, CH)]], rv, sm)

        return carry

    lax.fori_loop(0, K // 2, pair, 0)

    plsc.subcore_barrier()
    pltpu.sync_copy(acc.at[pl.ds(row0, RPT)],
                    out_hbm.at[cid, pl.ds(row0, RPT)])


_sc_scatter = functools.partial(
    pl.kernel,
    out_type=jax.ShapeDtypeStruct((NC, NP, D), jnp.float32),
    mesh=plsc.VectorSubcoreMesh(core_axis_name="c", subcore_axis_name="s"),
    scratch_types=[
        pltpu.VMEM((EPP,), jnp.int32),
        pltpu.VMEM((K, CH), jnp.int32),
        pltpu.VMEM((CH, D), jnp.float32),
        pltpu.VMEM((CH, D), jnp.float32),
        pltpu.VMEM_SHARED((NP, D), jnp.float32),
        pltpu.SemaphoreType.DMA,
        pltpu.SemaphoreType.DMA,
    ],
)(_sc_body)


def kernel(edge_index, graph_embedding, weight):
    ei = edge_index.astype(jnp.int32)
    pad = EPP - EPT
    src = jnp.pad(ei[0].reshape(NW, EPT), ((0, 0), (0, pad))).reshape(-1)
    dump = N_NODES + jnp.arange(NW, dtype=jnp.int32)
    dst = jnp.concatenate(
        [ei[1].reshape(NW, EPT),
         jnp.broadcast_to(dump[:, None], (NW, pad))], axis=1
    ).reshape(NW, K, CH)

    emb = pl.pallas_call(
        _elu_body,
        out_shape=jax.ShapeDtypeStruct((N_NODES, D), jnp.float32),
    )(graph_embedding, weight)

    partials = _sc_scatter(src, dst, emb)

    out = pl.pallas_call(
        _add_body,
        out_shape=jax.ShapeDtypeStruct((N_NODES, D), jnp.float32),
    )(partials)
    return out


# speedup vs baseline: 1.5684x; 1.5684x over previous
import functools

import jax
import jax.numpy as jnp
from jax import lax
from jax.experimental import pallas as pl
from jax.experimental.pallas import tpu as pltpu
from jax.experimental.pallas import tpu_sc as plsc

N_NODES = 10000
N_EDGES = 320000
D = 128
NC = 2
NS = 16
NW = NC * NS
EPT = N_EDGES // NW
CH = 100
K = EPT // CH
NP = 10240
RPT = NP // NS


def _elu_body(g_ref, w_ref, out_ref):
    x = g_ref[...] * w_ref[...]
    out_ref[...] = jnp.where(x > 0, x, jnp.exp(jnp.minimum(x, 0.0)) - 1.0)


def _add_body(p_ref, out_ref):
    out_ref[...] = p_ref[0, :N_NODES] + p_ref[1, :N_NODES]


def _sc_body(src_hbm, dst_hbm, emb_hbm, out_hbm, src_v, dst_v, rows, acc,
             sem):
    cid = lax.axis_index("c")
    sid = lax.axis_index("s")
    wid = cid * NS + sid

    def zstore(t, carry):
        i = t // (D // 16)
        j = t % (D // 16)
        rows[i, pl.ds(j * 16, 16)] = jnp.zeros((16,), jnp.float32)
        return carry

    lax.fori_loop(0, CH * (D // 16), zstore, 0)

    row0 = sid * RPT
    for r in range(RPT // CH):
        pltpu.sync_copy(rows, acc.at[pl.ds(row0 + r * CH, CH)])
    rem = RPT % CH
    if rem:
        pltpu.sync_copy(rows.at[pl.ds(0, rem)],
                        acc.at[pl.ds(row0 + RPT - rem, rem)])
    plsc.subcore_barrier()

    pltpu.sync_copy(src_hbm.at[wid], src_v)
    pltpu.sync_copy(dst_hbm.at[wid], dst_v)

    def chunk(j, carry):
        pltpu.async_copy(emb_hbm.at[src_v.at[j]], rows, sem).wait()
        pltpu.sync_copy(rows, acc.at[dst_v.at[j]], add=True)
        return carry

    lax.fori_loop(0, K, chunk, 0)

    plsc.subcore_barrier()
    pltpu.sync_copy(acc.at[pl.ds(row0, RPT)],
                    out_hbm.at[cid, pl.ds(row0, RPT)])


_sc_scatter = functools.partial(
    pl.kernel,
    out_type=jax.ShapeDtypeStruct((NC, NP, D), jnp.float32),
    mesh=plsc.VectorSubcoreMesh(core_axis_name="c", subcore_axis_name="s"),
    scratch_types=[
        pltpu.VMEM((K, CH), jnp.int32),
        pltpu.VMEM((K, CH), jnp.int32),
        pltpu.VMEM((CH, D), jnp.float32),
        pltpu.VMEM_SHARED((NP, D), jnp.float32),
        pltpu.SemaphoreType.DMA,
    ],
)(_sc_body)


def kernel(edge_index, graph_embedding, weight):
    ei = edge_index.astype(jnp.int32)
    src = ei[0].reshape(NW, K, CH)
    dst = ei[1].reshape(NW, K, CH)

    emb = pl.pallas_call(
        _elu_body,
        out_shape=jax.ShapeDtypeStruct((N_NODES, D), jnp.float32),
    )(graph_embedding, weight)

    partials = _sc_scatter(src, dst, emb)

    out = pl.pallas_call(
        _add_body,
        out_shape=jax.ShapeDtypeStruct((N_NODES, D), jnp.float32),
    )(partials)
    return out
